# Initial kernel scaffold; baseline (speedup 1.0000x reference)
import jax
import jax.numpy as jnp
from jax.experimental import pallas as pl


def kernel(text, emb_table, fc_w, fc_b):
    raise NotImplementedError("write your pallas kernel here")


# speedup vs baseline: 28.1979x; 28.1979x over previous
import functools

import jax
import jax.numpy as jnp
from jax import lax
from jax.experimental import pallas as pl
from jax.experimental.pallas import tpu as pltpu
from jax.experimental.pallas import tpu_sc as plsc

_VOCAB = 1000000
_EMBED = 32
_NUM_CLASS = 16
_MAX_LEN = 200
_BATCH = 4096

_TOTAL = _BATCH * _MAX_LEN
_NC, _NS = 2, 16
_NW = _NC * _NS
_PER_W = _TOTAL // _NW
_CHUNK = 1600
_NCHUNK = _PER_W // _CHUNK


def _gather_body(idx_hbm, table_hbm, out_hbm, idx_v, rows_v, sem):
    wid = lax.axis_index("s") * _NC + lax.axis_index("c")
    base = wid * _PER_W

    def body(i, _):
        off = base + i * _CHUNK
        pltpu.sync_copy(idx_hbm.at[pl.ds(off, _CHUNK)], idx_v)
        pltpu.async_copy(table_hbm.at[idx_v], rows_v, sem).wait()
        pltpu.sync_copy(rows_v, out_hbm.at[pl.ds(off, _CHUNK)])
        return 0

    lax.fori_loop(0, _NCHUNK, body, 0)


_sc_gather = functools.partial(
    pl.kernel,
    out_type=jax.ShapeDtypeStruct((_TOTAL, _EMBED), jnp.float32),
    mesh=plsc.VectorSubcoreMesh(core_axis_name="c", subcore_axis_name="s"),
    scratch_types=[
        pltpu.VMEM((_CHUNK,), jnp.int32),
        pltpu.VMEM((_CHUNK, _EMBED), jnp.float32),
        pltpu.SemaphoreType.DMA,
    ],
    compiler_params=pltpu.CompilerParams(use_tc_tiling_on_sc=False),
)(_gather_body)


def _mm_body(x_ref, w_ref, b_ref, o_ref):
    acc = lax.dot_general(
        x_ref[...], w_ref[...],
        (((1,), (1,)), ((), ())),
        preferred_element_type=jnp.float32,
    )
    o_ref[...] = acc + b_ref[...]


_BB = 256


def _tc_matmul(x, fc_w, fc_b2d):
    return pl.pallas_call(
        _mm_body,
        grid=(_BATCH // _BB,),
        in_specs=[
            pl.BlockSpec((_BB, _MAX_LEN * _EMBED), lambda i: (i, 0)),
            pl.BlockSpec((_NUM_CLASS, _MAX_LEN * _EMBED), lambda i: (0, 0)),
            pl.BlockSpec((1, _NUM_CLASS), lambda i: (0, 0)),
        ],
        out_specs=pl.BlockSpec((_BB, _NUM_CLASS), lambda i: (i, 0)),
        out_shape=jax.ShapeDtypeStruct((_BATCH, _NUM_CLASS), jnp.float32),
    )(x, fc_w, fc_b2d)


@jax.jit
def kernel(text, emb_table, fc_w, fc_b):
    flat_idx = text.reshape(_TOTAL).astype(jnp.int32)
    rows = _sc_gather(flat_idx, emb_table)
    x = rows.reshape(_BATCH, _MAX_LEN * _EMBED)
    return _tc_matmul(x, fc_w, fc_b.reshape(1, _NUM_CLASS))


# speedup vs baseline: 28.5984x; 1.0142x over previous
import functools

import jax
import jax.numpy as jnp
from jax import lax
from jax.experimental import pallas as pl
from jax.experimental.pallas import tpu as pltpu
from jax.experimental.pallas import tpu_sc as plsc

_VOCAB = 1000000
_EMBED = 32
_NUM_CLASS = 16
_MAX_LEN = 200
_BATCH = 4096

_TOTAL = _BATCH * _MAX_LEN
_NC, _NS = 2, 16
_NW = _NC * _NS
_PER_W = _TOTAL // _NW
_CHUNK = 1280
_NCHUNK = _PER_W // _CHUNK


def _gather_body(idx_hbm, table_hbm, out_hbm, idx_all, rows_v, gsem, osem):
    wid = lax.axis_index("s") * _NC + lax.axis_index("c")
    base = wid * _PER_W
    pltpu.sync_copy(idx_hbm.at[pl.ds(base, _PER_W)], idx_all)

    def g_copy(i):
        b = i % 2
        return pltpu.make_async_copy(
            table_hbm.at[idx_all.at[pl.ds(i * _CHUNK, _CHUNK)]],
            rows_v.at[b], gsem.at[b])

    def o_copy(i):
        b = i % 2
        return pltpu.make_async_copy(
            rows_v.at[b], out_hbm.at[pl.ds(base + i * _CHUNK, _CHUNK)],
            osem.at[b])

    g_copy(0).start()
    for i in range(_NCHUNK):
        g_copy(i).wait()
        if i + 1 < _NCHUNK:
            if i >= 1:
                o_copy(i - 1).wait()
            g_copy(i + 1).start()
        o_copy(i).start()
    o_copy(_NCHUNK - 2).wait()
    o_copy(_NCHUNK - 1).wait()


_sc_gather = functools.partial(
    pl.kernel,
    out_type=jax.ShapeDtypeStruct((_TOTAL, _EMBED), jnp.float32),
    mesh=plsc.VectorSubcoreMesh(core_axis_name="c", subcore_axis_name="s"),
    scratch_types=[
        pltpu.VMEM((_PER_W,), jnp.int32),
        pltpu.VMEM((2, _CHUNK, _EMBED), jnp.float32),
        pltpu.SemaphoreType.DMA((2,)),
        pltpu.SemaphoreType.DMA((2,)),
    ],
    compiler_params=pltpu.CompilerParams(use_tc_tiling_on_sc=False),
)(_gather_body)


def _mm_body(x_ref, w_ref, b_ref, o_ref):
    acc = lax.dot_general(
        x_ref[...], w_ref[...],
        (((1,), (1,)), ((), ())),
        preferred_element_type=jnp.float32,
    )
    o_ref[...] = acc + b_ref[...]


_BB = 256


def _tc_matmul(x, fc_w, fc_b2d):
    return pl.pallas_call(
        _mm_body,
        grid=(_BATCH // _BB,),
        in_specs=[
            pl.BlockSpec((_BB, _MAX_LEN * _EMBED), lambda i: (i, 0)),
            pl.BlockSpec((_NUM_CLASS, _MAX_LEN * _EMBED), lambda i: (0, 0)),
            pl.BlockSpec((1, _NUM_CLASS), lambda i: (0, 0)),
        ],
        out_specs=pl.BlockSpec((_BB, _NUM_CLASS), lambda i: (i, 0)),
        out_shape=jax.ShapeDtypeStruct((_BATCH, _NUM_CLASS), jnp.float32),
    )(x, fc_w, fc_b2d)


@jax.jit
def kernel(text, emb_table, fc_w, fc_b):
    flat_idx = text.reshape(_TOTAL).astype(jnp.int32)
    rows = _sc_gather(flat_idx, emb_table)
    x = rows.reshape(_BATCH, _MAX_LEN * _EMBED)
    return _tc_matmul(x, fc_w, fc_b.reshape(1, _NUM_CLASS))


# speedup vs baseline: 32.2109x; 1.1263x over previous
import functools

import jax
import jax.numpy as jnp
from jax import lax
from jax.experimental import pallas as pl
from jax.experimental.pallas import tpu as pltpu
from jax.experimental.pallas import tpu_sc as plsc

_VOCAB = 1000000
_EMBED = 32
_NUM_CLASS = 16
_MAX_LEN = 200
_BATCH = 4096

_TOTAL = _BATCH * _MAX_LEN
_NC, _NS = 2, 16
_NW = _NC * _NS
_PER_W = _TOTAL // _NW
_CHUNK = 1280
_NCHUNK = _PER_W // _CHUNK


def _gather_body(idx_hbm, table_hbm, out_hbm, idx_all, rows_v, gsem, osem):
    wid = lax.axis_index("s") * _NC + lax.axis_index("c")
    base = wid * _PER_W
    pltpu.sync_copy(idx_hbm.at[pl.ds(base, _PER_W)], idx_all)

    def g_copy(i):
        b = i % 2
        return pltpu.make_async_copy(
            table_hbm.at[idx_all.at[pl.ds(i * _CHUNK, _CHUNK)]],
            rows_v.at[b], gsem.at[b])

    def o_copy(i):
        b = i % 2
        return pltpu.make_async_copy(
            rows_v.at[b], out_hbm.at[pl.ds(base + i * _CHUNK, _CHUNK)],
            osem.at[b])

    g_copy(0).start()
    for i in range(_NCHUNK):
        g_copy(i).wait()
        if i + 1 < _NCHUNK:
            if i >= 1:
                o_copy(i - 1).wait()
            g_copy(i + 1).start()
        o_copy(i).start()
    o_copy(_NCHUNK - 2).wait()
    o_copy(_NCHUNK - 1).wait()


_sc_gather = functools.partial(
    pl.kernel,
    out_type=jax.ShapeDtypeStruct((_TOTAL, _EMBED), jnp.float32),
    mesh=plsc.VectorSubcoreMesh(core_axis_name="c", subcore_axis_name="s"),
    scratch_types=[
        pltpu.VMEM((_PER_W,), jnp.int32),
        pltpu.VMEM((2, _CHUNK, _EMBED), jnp.float32),
        pltpu.SemaphoreType.DMA((2,)),
        pltpu.SemaphoreType.DMA((2,)),
    ],
    compiler_params=pltpu.CompilerParams(use_tc_tiling_on_sc=False),
)(_gather_body)


_LB = 8
_I4 = _BATCH // 4


def _mm_body(x_ref, w_ref, b_ref, o_ref):
    @pl.when(pl.program_id(0) == 0)
    def _():
        o_ref[...] = jnp.broadcast_to(b_ref[...], (_I4, 4 * _NUM_CLASS))

    acc = lax.dot_general(
        x_ref[0], w_ref[0], (((1,), (0,)), ((), ())),
        preferred_element_type=jnp.float32,
    )
    for j in range(1, _LB):
        acc += lax.dot_general(
            x_ref[j], w_ref[j], (((1,), (0,)), ((), ())),
            preferred_element_type=jnp.float32,
        )
    o_ref[...] += acc


def _tc_matmul(x3, w4, b4):
    return pl.pallas_call(
        _mm_body,
        grid=(_MAX_LEN // _LB,),
        in_specs=[
            pl.BlockSpec((_LB, _I4, 128), lambda i: (i, 0, 0)),
            pl.BlockSpec((_LB, 128, 4 * _NUM_CLASS), lambda i: (i, 0, 0)),
            pl.BlockSpec((1, 4 * _NUM_CLASS), lambda i: (0, 0)),
        ],
        out_specs=pl.BlockSpec((_I4, 4 * _NUM_CLASS), lambda i: (0, 0)),
        out_shape=jax.ShapeDtypeStruct((_I4, 4 * _NUM_CLASS), jnp.float32),
    )(x3, w4, b4)


@jax.jit
def kernel(text, emb_table, fc_w, fc_b):
    flat_idx = text.T.reshape(_TOTAL).astype(jnp.int32)
    rows = _sc_gather(flat_idx, emb_table)
    x3 = rows.reshape(_MAX_LEN, _I4, 128)
    wl = fc_w.reshape(_NUM_CLASS, _MAX_LEN, _EMBED).transpose(1, 2, 0)
    eye4 = jnp.eye(4, dtype=jnp.float32)
    w4 = (wl[:, None, :, None, :] * eye4[None, :, None, :, None]).reshape(
        _MAX_LEN, 128, 4 * _NUM_CLASS)
    b4 = jnp.tile(fc_b, 4).reshape(1, 4 * _NUM_CLASS)
    out4 = _tc_matmul(x3, w4, b4)
    return out4.reshape(_BATCH, _NUM_CLASS)
